# baseline (speedup 1.0000x reference)
import functools

import jax
import jax.numpy as jnp
from jax import lax
from jax.experimental import pallas as pl
from jax.experimental.pallas import tpu as pltpu
from jax.experimental.pallas import tpu_sc as plsc

B = 100000
D = 200
NC = 2
NS = 16
NW = NC * NS
C = 128
NFULL = B // C
TAIL = B - NFULL * C
TAIL_WORKER = NFULL % NW

_mesh = plsc.VectorSubcoreMesh(core_axis_name="c", subcore_axis_name="s")


@functools.partial(
    pl.kernel,
    mesh=_mesh,
    compiler_params=pltpu.CompilerParams(use_tc_tiling_on_sc=False),
    out_type=jax.ShapeDtypeStruct((B, D), jnp.float32),
    scratch_types=[
        pltpu.VMEM((C,), jnp.int32),
        pltpu.VMEM((C, D), jnp.float32),
        pltpu.SemaphoreType.DMA,
    ],
)
def _gather_kernel(idx_hbm, table_hbm, out_hbm, idx_v, rows_v, sem):
    wid = lax.axis_index("s") * NC + lax.axis_index("c")

    def body(k, carry):
        chunk = wid + k * NW
        base = chunk * C
        pltpu.sync_copy(idx_hbm.at[pl.ds(base, C)], idx_v)
        pltpu.async_copy(table_hbm.at[idx_v], rows_v, sem).wait()
        pltpu.sync_copy(rows_v, out_hbm.at[pl.ds(base, C)])
        return carry

    nk = (NFULL - wid + NW - 1) // NW
    lax.fori_loop(0, nk, body, 0)

    @pl.when(wid == TAIL_WORKER)
    def _():
        base = NFULL * C
        pltpu.sync_copy(idx_hbm.at[pl.ds(base, TAIL)], idx_v.at[pl.ds(0, TAIL)])
        pltpu.async_copy(
            table_hbm.at[idx_v.at[pl.ds(0, TAIL)]],
            rows_v.at[pl.ds(0, TAIL)],
            sem,
        ).wait()
        pltpu.sync_copy(rows_v.at[pl.ds(0, TAIL)], out_hbm.at[pl.ds(base, TAIL)])


def kernel(x, atom_fea):
    return _gather_kernel(x.astype(jnp.int32), atom_fea)
